# baseline (speedup 1.0000x reference)
import functools
import math

import jax
import jax.numpy as jnp
from jax import lax
from jax.experimental import pallas as pl
from jax.experimental.pallas import tpu as pltpu
from jax.experimental.pallas import tpu_sc as plsc

NC = 2
NS = 16
NW = NC * NS
LANES = 16
CHUNK = 128


def _gather_scaled(tokens_3d, table, *, nchunk, scale):
    V, D = table.shape
    B = NW * nchunk * CHUNK
    mesh = plsc.VectorSubcoreMesh(
        core_axis_name="c", subcore_axis_name="s",
        num_cores=NC, num_subcores=NS)

    @functools.partial(
        pl.kernel,
        out_type=jax.ShapeDtypeStruct((B, D), jnp.float32),
        mesh=mesh,
        scratch_types=[
            pltpu.VMEM((nchunk, CHUNK), jnp.int32),
            pltpu.VMEM((CHUNK, D), jnp.float32),
            pltpu.SemaphoreType.DMA,
        ],
        compiler_params=pltpu.CompilerParams(use_tc_tiling_on_sc=False),
    )
    def k(tokens_hbm, table_hbm, out_hbm, idx_v, rows_v, gsem):
        wid = lax.axis_index("s") * NC + lax.axis_index("c")
        base = wid * (nchunk * CHUNK)
        pltpu.sync_copy(tokens_hbm.at[wid], idx_v)

        @pl.loop(0, nchunk)
        def _chunk(j):
            pltpu.async_copy(table_hbm.at[idx_v.at[j]], rows_v, gsem).wait()

            @pl.loop(0, CHUNK)
            def _row(r):
                for c in range(D // LANES):
                    sl = pl.ds(c * LANES, LANES)
                    rows_v[r, sl] = rows_v[r, sl] * scale

            pltpu.sync_copy(rows_v, out_hbm.at[pl.ds(base + j * CHUNK, CHUNK)])

    return k(tokens_3d, table)


def kernel(tokens, embedding_weight):
    B0, S = tokens.shape
    V, D = embedding_weight.shape
    B = B0 * S
    assert B % (NW * CHUNK) == 0 and D % LANES == 0
    nchunk = B // (NW * CHUNK)
    scale = math.sqrt(D)
    flat = tokens.reshape(NW, nchunk, CHUNK).astype(jnp.int32)
    out = _gather_scaled(flat, embedding_weight, nchunk=nchunk, scale=scale)
    return out.reshape(B0, S, D)


# speedup vs baseline: 1.2079x; 1.2079x over previous
import functools
import math

import jax
import jax.numpy as jnp
from jax import lax
from jax.experimental import pallas as pl
from jax.experimental.pallas import tpu as pltpu
from jax.experimental.pallas import tpu_sc as plsc

NC = 2
NS = 16
NW = NC * NS
LANES = 16
CHUNK = 128
NBUF = 8


def _gather_scaled(tokens_3d, table, *, nchunk, scale):
    V, D = table.shape
    B = NW * nchunk * CHUNK
    assert nchunk % NBUF == 0 and nchunk >= 2 * NBUF
    mesh = plsc.VectorSubcoreMesh(
        core_axis_name="c", subcore_axis_name="s",
        num_cores=NC, num_subcores=NS)

    @functools.partial(
        pl.kernel,
        out_type=jax.ShapeDtypeStruct((B, D), jnp.float32),
        mesh=mesh,
        scratch_types=[
            pltpu.VMEM((nchunk, CHUNK), jnp.int32),
            *([pltpu.VMEM((CHUNK, D), jnp.float32)] * NBUF),
            *([pltpu.SemaphoreType.DMA] * (2 * NBUF)),
        ],
        compiler_params=pltpu.CompilerParams(use_tc_tiling_on_sc=False),
    )
    def k(tokens_hbm, table_hbm, out_hbm, idx_v, *bufs_and_sems):
        rows = bufs_and_sems[:NBUF]
        gsem = bufs_and_sems[NBUF:2 * NBUF]
        ssem = bufs_and_sems[2 * NBUF:]
        wid = lax.axis_index("s") * NC + lax.axis_index("c")
        base = wid * (nchunk * CHUNK)
        pltpu.sync_copy(tokens_hbm.at[wid], idx_v)

        for b in range(NBUF):
            pltpu.async_copy(table_hbm.at[idx_v.at[b]], rows[b], gsem[b])

        @pl.loop(0, nchunk, step=NBUF)
        def _group(j0):
            for b in range(NBUF):
                j = j0 + b
                pltpu.make_async_copy(
                    table_hbm.at[idx_v.at[j]], rows[b], gsem[b]).wait()

                @pl.loop(0, CHUNK, unroll=8)
                def _row(r):
                    for c in range(D // LANES):
                        sl = pl.ds(c * LANES, LANES)
                        rows[b][r, sl] = rows[b][r, sl] * scale

                out_slice = out_hbm.at[pl.ds(base + j * CHUNK, CHUNK)]
                pltpu.async_copy(rows[b], out_slice, ssem[b])

                br = (b - 2) % NBUF
                jr = j - 2 + NBUF

                @pl.when(jnp.logical_and(jr >= NBUF, jr < nchunk))
                def _refill():
                    prev = out_hbm.at[pl.ds(base + (jr - NBUF) * CHUNK, CHUNK)]
                    pltpu.make_async_copy(rows[br], prev, ssem[br]).wait()
                    pltpu.async_copy(
                        table_hbm.at[idx_v.at[jr]], rows[br], gsem[br])

        for b in range(NBUF):
            j = nchunk - NBUF + b
            out_slice = out_hbm.at[pl.ds(base + j * CHUNK, CHUNK)]
            pltpu.make_async_copy(rows[b], out_slice, ssem[b]).wait()

    return k(tokens_3d, table)


def kernel(tokens, embedding_weight):
    B0, S = tokens.shape
    V, D = embedding_weight.shape
    B = B0 * S
    assert B % (NW * CHUNK) == 0 and D % LANES == 0
    nchunk = B // (NW * CHUNK)
    scale = math.sqrt(D)
    flat = tokens.reshape(NW, nchunk, CHUNK).astype(jnp.int32)
    out = _gather_scaled(flat, embedding_weight, nchunk=nchunk, scale=scale)
    return out.reshape(B0, S, D)
